# Initial kernel scaffold; baseline (speedup 1.0000x reference)
import jax
import jax.numpy as jnp
from jax.experimental import pallas as pl


def kernel(x, edge_index, batch, edge_attr, params):
    raise NotImplementedError("write your pallas kernel here")


# speedup vs baseline: 1.4840x; 1.4840x over previous
import functools

import jax
import jax.numpy as jnp
from jax import lax
from jax.experimental import pallas as pl

N_NODES = 50000
N_EDGES = 1600000
NUM_GRAPHS = 128
FEATS_DIM = 5
POS_DIM = 3
M_DIM = 16

BE = 2000
BN = 2000
GE = N_EDGES // BE
GN = N_NODES // BN


def _silu(v):
    return v * jax.nn.sigmoid(v)


def _edge_body(xs_ref, xd_ref, ea_ref, w1t_ref, b1_ref, w2t_ref, b2_ref,
               swt_ref, sb_ref, cw1t_ref, cb1_ref, cw2t_ref, cb2_ref,
               cs_ref, msg_ref, wv_ref):
    xs = xs_ref[...]
    xd = xd_ref[...]
    ea = ea_ref[...]
    rel = xs[:, 0:POS_DIM] - xd[:, 0:POS_DIM]
    rd = jnp.sum(rel * rel, axis=1, keepdims=True)
    m_in = jnp.concatenate([xd[:, POS_DIM:], xs[:, POS_DIM:], ea, rd], axis=1)
    h = _silu(jnp.dot(m_in, w1t_ref[...], preferred_element_type=jnp.float32)
              + b1_ref[...])
    mij = _silu(jnp.dot(h, w2t_ref[...], preferred_element_type=jnp.float32)
                + b2_ref[...])
    ch = _silu(jnp.dot(mij, cw1t_ref[...], preferred_element_type=jnp.float32)
               + cb1_ref[...])
    cwij = (jnp.dot(ch, cw2t_ref[...], preferred_element_type=jnp.float32)
            + cb2_ref[...])
    nrm = jnp.sqrt(jnp.maximum(rd, 1e-16))
    reln = rel / jnp.maximum(nrm, 1e-8) * cs_ref[0, 0]
    wv = cwij * reln
    gate = jax.nn.sigmoid(
        jnp.dot(mij, swt_ref[...], preferred_element_type=jnp.float32)
        + sb_ref[...])
    msg_ref[...] = mij * gate
    wv_ref[...] = jnp.concatenate(
        [wv, jnp.zeros((wv.shape[0], 8 - POS_DIM), jnp.float32)], axis=1)


def _edge_call(xs, xd, ea, p):
    full = lambda shp: pl.BlockSpec(shp, lambda i: (0, 0))
    return pl.pallas_call(
        _edge_body,
        grid=(GE,),
        in_specs=[
            pl.BlockSpec((BE, 8), lambda i: (i, 0)),
            pl.BlockSpec((BE, 8), lambda i: (i, 0)),
            pl.BlockSpec((BE, 4), lambda i: (i, 0)),
            full((15, 30)), full((1, 30)),
            full((30, 16)), full((1, 16)),
            full((16, 1)), full((1, 1)),
            full((16, 64)), full((1, 64)),
            full((64, 1)), full((1, 1)),
            full((1, 1)),
        ],
        out_specs=[
            pl.BlockSpec((BE, 16), lambda i: (i, 0)),
            pl.BlockSpec((BE, 8), lambda i: (i, 0)),
        ],
        out_shape=[
            jax.ShapeDtypeStruct((N_EDGES, 16), jnp.float32),
            jax.ShapeDtypeStruct((N_EDGES, 8), jnp.float32),
        ],
    )(xs, xd, ea,
      p["edge_w1"].T, p["edge_b1"][None, :],
      p["edge_w2"].T, p["edge_b2"][None, :],
      p["soft_w"].T, p["soft_b"][None, :],
      p["coors_w1"].T, p["coors_b1"][None, :],
      p["coors_w2"].T, p["coors_b2"][None, :],
      p["coors_scale"].reshape(1, 1))


def _onehot(b):
    g = lax.broadcasted_iota(jnp.int32, (b.shape[0], NUM_GRAPHS), 1)
    return (b == g).astype(jnp.float32)


def _segdot(oh, vals):
    return lax.dot_general(oh, vals, (((0,), (0,)), ((), ())),
                           preferred_element_type=jnp.float32)


def _ns_body(x_ref, b_ref, out_ref):
    @pl.when(pl.program_id(0) == 0)
    def _():
        out_ref[...] = jnp.zeros_like(out_ref)

    feats = x_ref[...][:, POS_DIM:]
    v1 = jnp.sum(feats, axis=1, keepdims=True)
    v2 = jnp.sum(feats * feats, axis=1, keepdims=True)
    ones = jnp.ones_like(v1)
    vals = jnp.concatenate(
        [v1, v2, ones, jnp.zeros((v1.shape[0], 5), jnp.float32)], axis=1)
    out_ref[...] += _segdot(_onehot(b_ref[...]), vals)


def _ns_call(x, b2):
    return pl.pallas_call(
        _ns_body,
        grid=(GN,),
        in_specs=[
            pl.BlockSpec((BN, 8), lambda i: (i, 0)),
            pl.BlockSpec((BN, 1), lambda i: (i, 0)),
        ],
        out_specs=pl.BlockSpec((NUM_GRAPHS, 8), lambda i: (0, 0)),
        out_shape=jax.ShapeDtypeStruct((NUM_GRAPHS, 8), jnp.float32),
    )(x, b2)


def _nu_body(x_ref, am_ref, aw_ref, b_ref, ls_ref, lnw_ref, lnb_ref,
             nw1t_ref, nb1_ref, nw2t_ref, nb2_ref, xp_ref, gs_ref):
    x = x_ref[...]
    m_i = jnp.sum(am_ref[...], axis=0)
    mw = jnp.sum(aw_ref[...], axis=0)
    coors = x[:, 0:POS_DIM] + mw[:, 0:POS_DIM]
    feats = x[:, POS_DIM:]

    ls = ls_ref[...]
    normv = jnp.maximum(ls[:, 2:3], 1.0) * float(FEATS_DIM)
    m = ls[:, 0:1] / normv
    var = ls[:, 1:2] / normv - m * m
    inv = lax.rsqrt(var + 1e-5)
    pg = jnp.concatenate(
        [m, inv, jnp.zeros((NUM_GRAPHS, 6), jnp.float32)], axis=1)
    pn = jnp.dot(_onehot(b_ref[...]), pg, preferred_element_type=jnp.float32)
    feats_n = (feats - pn[:, 0:1]) * pn[:, 1:2] * lnw_ref[...] + lnb_ref[...]

    h2in = jnp.concatenate([feats_n, m_i], axis=1)
    h2 = _silu(jnp.dot(h2in, nw1t_ref[...], preferred_element_type=jnp.float32)
               + nb1_ref[...])
    fo = feats + jnp.dot(h2, nw2t_ref[...],
                         preferred_element_type=jnp.float32) + nb2_ref[...]
    xp = jnp.concatenate([coors, fo], axis=1)
    xp_ref[...] = xp

    @pl.when(pl.program_id(0) == 0)
    def _():
        gs_ref[...] = jnp.zeros_like(gs_ref)

    s1 = jnp.sum(xp, axis=0, keepdims=True)
    s2 = jnp.sum(xp * xp, axis=0, keepdims=True)
    gs_ref[...] += jnp.concatenate(
        [s1, s2, jnp.zeros((6, 8), jnp.float32)], axis=0)


def _nu_call(x, accm, accw, b2, lnstats, p):
    full = lambda shp: pl.BlockSpec(shp, lambda i: tuple(0 for _ in shp))
    a = accm.shape[0]
    return pl.pallas_call(
        _nu_body,
        grid=(GN,),
        in_specs=[
            pl.BlockSpec((BN, 8), lambda i: (i, 0)),
            pl.BlockSpec((a, BN, 16), lambda i: (0, i, 0)),
            pl.BlockSpec((a, BN, 8), lambda i: (0, i, 0)),
            pl.BlockSpec((BN, 1), lambda i: (i, 0)),
            full((NUM_GRAPHS, 8)),
            full((1, 5)), full((1, 5)),
            full((21, 10)), full((1, 10)),
            full((10, 5)), full((1, 5)),
        ],
        out_specs=[
            pl.BlockSpec((BN, 8), lambda i: (i, 0)),
            pl.BlockSpec((8, 8), lambda i: (0, 0)),
        ],
        out_shape=[
            jax.ShapeDtypeStruct((N_NODES, 8), jnp.float32),
            jax.ShapeDtypeStruct((8, 8), jnp.float32),
        ],
    )(x, accm, accw, b2, lnstats,
      p["ln_w"][None, :], p["ln_b"][None, :],
      p["node_w1"].T, p["node_b1"][None, :],
      p["node_w2"].T, p["node_b2"][None, :])


def _ga_body(xp_ref, gs_ref, b_ref, gw_ref, gb_ref, gms_ref,
             y_ref, st_ref, *, relu, last):
    gs = gs_ref[...]
    nf = float(N_NODES)
    mean = gs[0:1, :] / nf
    e2 = gs[1:2, :] / nf
    c = mean * gms_ref[...]
    var = e2 - 2.0 * mean * c + c * c
    y = gw_ref[...] * (xp_ref[...] - c) / jnp.sqrt(var + 1e-5) + gb_ref[...]
    if relu:
        y = jnp.maximum(y, 0.0)
    y_ref[...] = y

    @pl.when(pl.program_id(0) == 0)
    def _():
        st_ref[...] = jnp.zeros_like(st_ref)

    oh = _onehot(b_ref[...])
    if last:
        st_ref[...] += _segdot(oh, y)
    else:
        feats = y[:, POS_DIM:]
        v1 = jnp.sum(feats, axis=1, keepdims=True)
        v2 = jnp.sum(feats * feats, axis=1, keepdims=True)
        ones = jnp.ones_like(v1)
        vals = jnp.concatenate(
            [v1, v2, ones, jnp.zeros((v1.shape[0], 5), jnp.float32)], axis=1)
        st_ref[...] += _segdot(oh, vals)


def _ga_call(xp, gstats, b2, g, relu, last):
    full = lambda shp: pl.BlockSpec(shp, lambda i: (0, 0))
    return pl.pallas_call(
        functools.partial(_ga_body, relu=relu, last=last),
        grid=(GN,),
        in_specs=[
            pl.BlockSpec((BN, 8), lambda i: (i, 0)),
            full((8, 8)),
            pl.BlockSpec((BN, 1), lambda i: (i, 0)),
            full((1, 8)), full((1, 8)), full((1, 8)),
        ],
        out_specs=[
            pl.BlockSpec((BN, 8), lambda i: (i, 0)),
            pl.BlockSpec((NUM_GRAPHS, 8), lambda i: (0, 0)),
        ],
        out_shape=[
            jax.ShapeDtypeStruct((N_NODES, 8), jnp.float32),
            jax.ShapeDtypeStruct((NUM_GRAPHS, 8), jnp.float32),
        ],
    )(xp, gstats, b2,
      g["weight"][None, :], g["bias"][None, :], g["mean_scale"][None, :])


def _head_body(pool_ref, cnt_ref, w1t_ref, b1_ref, w2t_ref, b2_ref, out_ref):
    h = pool_ref[...] / jnp.maximum(cnt_ref[...], 1.0)
    h1 = jnp.maximum(
        jnp.dot(h, w1t_ref[...], preferred_element_type=jnp.float32)
        + b1_ref[...], 0.0)
    out_ref[...] = (jnp.dot(h1, w2t_ref[...],
                            preferred_element_type=jnp.float32) + b2_ref[...])


def _head_call(pool, cnt, fc):
    (w1, b1), (w2, b2) = fc
    full = lambda shp: pl.BlockSpec(shp, lambda: (0, 0))
    return pl.pallas_call(
        _head_body,
        in_specs=[full((NUM_GRAPHS, 8)), full((NUM_GRAPHS, 1)),
                  full((8, 32)), full((1, 32)),
                  full((32, 10)), full((1, 10))],
        out_specs=full((NUM_GRAPHS, 10)),
        out_shape=jax.ShapeDtypeStruct((NUM_GRAPHS, 10), jnp.float32),
    )(pool, cnt, w1.T, b1[None, :], w2.T, b2[None, :])


def kernel(x, edge_index, batch, edge_attr, params):
    src = edge_index[0]
    dst = edge_index[1]
    b2 = batch[:, None]

    lnstats = _ns_call(x, b2)
    cnt = lnstats[:, 2:3]

    x_cur = x
    for i in range(3):
        p = params["layers"][i]
        xs = jnp.take(x_cur, src, axis=0)
        xd = jnp.take(x_cur, dst, axis=0)
        msg, wv = _edge_call(xs, xd, edge_attr, p)
        accm = jax.ops.segment_sum(msg, dst, num_segments=N_NODES)[None]
        accw = jax.ops.segment_sum(wv, dst, num_segments=N_NODES)[None]
        xp, gstats = _nu_call(x_cur, accm, accw, b2, lnstats, p)
        last = i == 2
        x_cur, aux = _ga_call(xp, gstats, b2, params["gn"][i],
                              relu=not last, last=last)
        if not last:
            lnstats = aux
    return _head_call(aux, cnt, params["fc"])
